# baseline (speedup 1.0000x reference)
import jax
import jax.numpy as jnp
from jax import lax
from jax.experimental import pallas as pl
from jax.experimental.pallas import tpu as pltpu

_F = 4
_H = 16
_LANES = 128
_SPR = _LANES // _F
_RATIO = _H * _SPR // _LANES
_BLOCK_ROWS = 2048


def _fused_body(x_ref, w_ref, b_ref, o_ref):
    bt = x_ref.shape[0]
    y = lax.dot_general(
        x_ref[...], w_ref[...],
        dimension_numbers=(((1,), (0,)), ((), ())),
        preferred_element_type=jnp.float32,
    )
    y = jnp.maximum(y + b_ref[...], 0.0)
    for k in range(_RATIO):
        o_ref[pl.ds(k, bt, stride=_RATIO), :] = y[:, k * _LANES:(k + 1) * _LANES]


def kernel(x, w1, b1):
    B = x.shape[0]
    block = _BLOCK_ROWS
    samples_per_block = _SPR * block
    if B % samples_per_block:
        Bp = -(-B // samples_per_block) * samples_per_block
        x = jnp.pad(x, ((0, Bp - B), (0, 0)))
    else:
        Bp = B
    rows = Bp // _SPR
    x2 = x.reshape(rows, _LANES)

    w_cat = (jnp.eye(_SPR, dtype=jnp.float32)[:, None, :, None]
             * w1.T[None, :, None, :]).reshape(_LANES, _SPR * _H)
    b_cat = jnp.tile(b1.reshape(1, _H), (1, _SPR))

    out2 = pl.pallas_call(
        _fused_body,
        out_shape=jax.ShapeDtypeStruct((_RATIO * rows, _LANES), jnp.float32),
        grid=(rows // block,),
        in_specs=[
            pl.BlockSpec((block, _LANES), lambda i: (i, 0)),
            pl.BlockSpec((_LANES, _SPR * _H), lambda i: (0, 0)),
            pl.BlockSpec((1, _SPR * _H), lambda i: (0, 0)),
        ],
        out_specs=pl.BlockSpec((_RATIO * block, _LANES), lambda i: (i, 0)),
        compiler_params=pltpu.CompilerParams(
            dimension_semantics=("parallel",),
        ),
    )(x2, w_cat, b_cat)

    out = out2.reshape(Bp, _H)
    return out if Bp == B else out[:B]


# speedup vs baseline: 63.2815x; 63.2815x over previous
import jax
import jax.numpy as jnp
from jax import lax
from jax.experimental import pallas as pl
from jax.experimental.pallas import tpu as pltpu

_F = 4
_H = 16
_BLOCK = 65536


def _mlp_t_body(x_ref, w_ref, b_ref, o_ref):
    y = lax.dot_general(
        w_ref[...], x_ref[...],
        dimension_numbers=(((1,), (0,)), ((), ())),
        preferred_element_type=jnp.float32,
    )
    o_ref[...] = jnp.maximum(y + b_ref[...], 0.0)


def kernel(x, w1, b1):
    B = x.shape[0]
    bt = _BLOCK
    if B % bt:
        Bp = -(-B // bt) * bt
        x = jnp.pad(x, ((0, Bp - B), (0, 0)))
    else:
        Bp = B

    xt = x.T
    bcol = b1.T

    out_t = pl.pallas_call(
        _mlp_t_body,
        out_shape=jax.ShapeDtypeStruct((_H, Bp), jnp.float32),
        grid=(Bp // bt,),
        in_specs=[
            pl.BlockSpec((_F, bt), lambda i: (0, i)),
            pl.BlockSpec((_H, _F), lambda i: (0, 0)),
            pl.BlockSpec((_H, 1), lambda i: (0, 0)),
        ],
        out_specs=pl.BlockSpec((_H, bt), lambda i: (0, i)),
        compiler_params=pltpu.CompilerParams(
            dimension_semantics=("parallel",),
        ),
    )(xt, w1, bcol)

    out = out_t.T
    return out if Bp == B else out[:B]


# speedup vs baseline: 69.0408x; 1.0910x over previous
import jax
import jax.numpy as jnp
from jax import lax
from jax.experimental import pallas as pl
from jax.experimental.pallas import tpu as pltpu

_F = 4
_H = 16
_BLOCK = 131072


def _mlp_t_body(x_ref, w_ref, b_ref, o_ref):
    y = lax.dot_general(
        w_ref[...], x_ref[...],
        dimension_numbers=(((1,), (0,)), ((), ())),
        preferred_element_type=jnp.float32,
    )
    o_ref[...] = jnp.maximum(y + b_ref[...], 0.0)


def kernel(x, w1, b1):
    B = x.shape[0]
    bt = _BLOCK
    if B % bt:
        Bp = -(-B // bt) * bt
        x = jnp.pad(x, ((0, Bp - B), (0, 0)))
    else:
        Bp = B

    xt = x.T
    bcol = b1.T

    out_t = pl.pallas_call(
        _mlp_t_body,
        out_shape=jax.ShapeDtypeStruct((_H, Bp), jnp.float32),
        grid=(Bp // bt,),
        in_specs=[
            pl.BlockSpec((_F, bt), lambda i: (0, i)),
            pl.BlockSpec((_H, _F), lambda i: (0, 0)),
            pl.BlockSpec((_H, 1), lambda i: (0, 0)),
        ],
        out_specs=pl.BlockSpec((_H, bt), lambda i: (0, i)),
        compiler_params=pltpu.CompilerParams(
            dimension_semantics=("parallel",),
        ),
    )(xt, w1, bcol)

    out = out_t.T
    return out if Bp == B else out[:B]


# speedup vs baseline: 70.1794x; 1.0165x over previous
import jax
import jax.numpy as jnp
from jax import lax
from jax.experimental import pallas as pl
from jax.experimental.pallas import tpu as pltpu

_F = 4
_H = 16
_BLOCK = 262144


def _mlp_t_body(x_ref, w_ref, b_ref, o_ref):
    y = lax.dot_general(
        w_ref[...], x_ref[...],
        dimension_numbers=(((1,), (0,)), ((), ())),
        preferred_element_type=jnp.float32,
    )
    o_ref[...] = jnp.maximum(y + b_ref[...], 0.0)


def kernel(x, w1, b1):
    B = x.shape[0]
    bt = _BLOCK
    if B % bt:
        Bp = -(-B // bt) * bt
        x = jnp.pad(x, ((0, Bp - B), (0, 0)))
    else:
        Bp = B

    xt = x.T
    bcol = b1.T

    out_t = pl.pallas_call(
        _mlp_t_body,
        out_shape=jax.ShapeDtypeStruct((_H, Bp), jnp.float32),
        grid=(Bp // bt,),
        in_specs=[
            pl.BlockSpec((_F, bt), lambda i: (0, i)),
            pl.BlockSpec((_H, _F), lambda i: (0, 0)),
            pl.BlockSpec((_H, 1), lambda i: (0, 0)),
        ],
        out_specs=pl.BlockSpec((_H, bt), lambda i: (0, i)),
        compiler_params=pltpu.CompilerParams(
            dimension_semantics=("parallel",),
        ),
    )(xt, w1, bcol)

    out = out_t.T
    return out if Bp == B else out[:B]


# speedup vs baseline: 73.7104x; 1.0503x over previous
import jax
import jax.numpy as jnp
from jax import lax
from jax.experimental import pallas as pl
from jax.experimental.pallas import tpu as pltpu

_F = 4
_H = 16
_BLOCK = 262144


def _mlp_t_body(x_ref, wt_ref, b_ref, o_ref):
    y = lax.dot_general(
        jnp.transpose(wt_ref[...]), x_ref[...],
        dimension_numbers=(((1,), (0,)), ((), ())),
        preferred_element_type=jnp.float32,
    )
    bcol = jnp.transpose(b_ref[...])
    o_ref[...] = jnp.maximum(y + bcol, 0.0)


def kernel(x, w1, b1):
    B = x.shape[0]
    bt = _BLOCK
    if B % bt:
        Bp = -(-B // bt) * bt
        x = jnp.pad(x, ((0, Bp - B), (0, 0)))
    else:
        Bp = B

    xt = x.T
    wt = w1.T

    out_t = pl.pallas_call(
        _mlp_t_body,
        out_shape=jax.ShapeDtypeStruct((_H, Bp), jnp.float32),
        grid=(Bp // bt,),
        in_specs=[
            pl.BlockSpec((_F, bt), lambda i: (0, i)),
            pl.BlockSpec((_F, _H), lambda i: (0, 0)),
            pl.BlockSpec((1, _H), lambda i: (0, 0)),
        ],
        out_specs=pl.BlockSpec((_H, bt), lambda i: (0, i)),
        compiler_params=pltpu.CompilerParams(
            dimension_semantics=("parallel",),
        ),
    )(xt, wt, b1)

    out = out_t.T
    return out if Bp == B else out[:B]
